# baseline (speedup 1.0000x reference)
import jax
import jax.numpy as jnp
from jax.experimental import pallas as pl
from jax.experimental.pallas import tpu as pltpu

_K = 4
_BLOCK_N = 2048


def _mlp_softmax_kernel(x0, x1, x2, x3, w1_ref, b1_ref, w2_ref, b2_ref,
                        o0, o1, o2, o3):
    w1 = w1_ref[...]
    b1 = b1_ref[...]
    w2 = w2_ref[...]
    b2 = b2_ref[...]
    for x_ref, o_ref in ((x0, o0), (x1, o1), (x2, o2), (x3, o3)):
        x = x_ref[...]
        ht = jax.lax.dot_general(
            w1, x, (((0,), (1,)), ((), ())),
            preferred_element_type=jnp.float32,
        ) + b1
        ht = 0.5 * ht * (1.0 + jax.lax.erf(ht * 0.7071067811865476))
        lt = jax.lax.dot_general(
            w2, ht, (((0,), (0,)), ((), ())),
            preferred_element_type=jnp.float32,
        ) + b2
        m = jnp.max(lt, axis=0, keepdims=True)
        e = jnp.exp(lt - m)
        p = e / jnp.sum(e, axis=0, keepdims=True)
        o_ref[...] = p.T


def kernel(batch_x, W1, b1, W2, b2):
    n, d = batch_x.shape
    h_dim = W1.shape[1]
    q = W2.shape[1]
    ns = n // _K
    steps = ns // _BLOCK_N
    grid = (steps,)

    def x_spec(k):
        return pl.BlockSpec(
            (_BLOCK_N, d), lambda i, k=k: (k * steps + i, 0)
        )

    out_specs = [
        pl.BlockSpec((_BLOCK_N, q), lambda i: (i, 0)) for _ in range(_K)
    ]
    out_shapes = [
        jax.ShapeDtypeStruct((ns, q), jnp.float32) for _ in range(_K)
    ]

    outs = pl.pallas_call(
        _mlp_softmax_kernel,
        grid=grid,
        in_specs=[
            x_spec(0), x_spec(1), x_spec(2), x_spec(3),
            pl.BlockSpec((d, h_dim), lambda i: (0, 0)),
            pl.BlockSpec((h_dim, 1), lambda i: (0, 0)),
            pl.BlockSpec((h_dim, q), lambda i: (0, 0)),
            pl.BlockSpec((q, 1), lambda i: (0, 0)),
        ],
        out_specs=out_specs,
        out_shape=out_shapes,
        compiler_params=pltpu.CompilerParams(
            dimension_semantics=("parallel",),
        ),
    )(batch_x, batch_x, batch_x, batch_x,
      W1, b1.reshape(h_dim, 1), W2, b2.reshape(q, 1))
    return jnp.concatenate(outs, axis=0)


# speedup vs baseline: 1.0261x; 1.0261x over previous
import jax
import jax.numpy as jnp
from jax.experimental import pallas as pl
from jax.experimental.pallas import tpu as pltpu

_BLOCK_N = 8192


def _mlp_block(x_ref, w1_ref, b1_ref, w2_ref, b2_ref, o_ref):
    x = x_ref[...]
    ht = jax.lax.dot_general(
        w1_ref[...], x, (((0,), (1,)), ((), ())),
        preferred_element_type=jnp.float32,
    ) + b1_ref[...]
    ht = 0.5 * ht * (1.0 + jax.lax.erf(ht * 0.7071067811865476))
    lt = jax.lax.dot_general(
        w2_ref[...], ht, (((0,), (0,)), ((), ())),
        preferred_element_type=jnp.float32,
    ) + b2_ref[...]
    m = jnp.max(lt, axis=0, keepdims=True)
    e = jnp.exp(lt - m)
    p = e / jnp.sum(e, axis=0, keepdims=True)
    o_ref[...] = p.T


def kernel(batch_x, W1, b1, W2, b2):
    n, d = batch_x.shape
    h_dim = W1.shape[1]
    q = W2.shape[1]
    steps = n // _BLOCK_N
    mesh = pltpu.create_tensorcore_mesh("core")

    def body(x_hbm, w1_hbm, b1_hbm, w2_hbm, b2_hbm, o_hbm):
        pltpu.emit_pipeline(
            _mlp_block,
            grid=(steps,),
            in_specs=[
                pl.BlockSpec((_BLOCK_N, d), lambda i: (i, 0)),
                pl.BlockSpec((d, h_dim), lambda i: (0, 0)),
                pl.BlockSpec((h_dim, 1), lambda i: (0, 0)),
                pl.BlockSpec((h_dim, q), lambda i: (0, 0)),
                pl.BlockSpec((q, 1), lambda i: (0, 0)),
            ],
            out_specs=[pl.BlockSpec((_BLOCK_N, q), lambda i: (i, 0))],
            core_axis_name="core",
            dimension_semantics=(pltpu.PARALLEL,),
        )(x_hbm, w1_hbm, b1_hbm, w2_hbm, b2_hbm, o_hbm)

    return pl.kernel(
        body,
        out_type=jax.ShapeDtypeStruct((n, q), jnp.float32),
        mesh=mesh,
    )(batch_x, W1, b1.reshape(h_dim, 1), W2, b2.reshape(q, 1))


# speedup vs baseline: 1.0667x; 1.0395x over previous
import jax
import jax.numpy as jnp
from jax.experimental import pallas as pl
from jax.experimental.pallas import tpu as pltpu

_BLOCK_N = 4096


def _mlp_block(x_ref, w1_ref, b1_ref, w2_ref, b2_ref, o_ref):
    x = x_ref[...]
    ht = jax.lax.dot_general(
        w1_ref[...], x, (((0,), (1,)), ((), ())),
        preferred_element_type=jnp.float32,
    ) + b1_ref[...]
    ht = 0.5 * ht * (1.0 + jax.lax.erf(ht * 0.7071067811865476))
    lt = jax.lax.dot_general(
        w2_ref[...], ht, (((0,), (0,)), ((), ())),
        preferred_element_type=jnp.float32,
    ) + b2_ref[...]
    m = jnp.max(lt, axis=0, keepdims=True)
    e = jnp.exp(lt - m)
    p = e / jnp.sum(e, axis=0, keepdims=True)
    o_ref[...] = p.T


def kernel(batch_x, W1, b1, W2, b2):
    n, d = batch_x.shape
    h_dim = W1.shape[1]
    q = W2.shape[1]
    steps = n // _BLOCK_N
    mesh = pltpu.create_tensorcore_mesh("core")

    def body(x_hbm, w1_hbm, b1_hbm, w2_hbm, b2_hbm, o_hbm):
        pltpu.emit_pipeline(
            _mlp_block,
            grid=(steps,),
            in_specs=[
                pl.BlockSpec((_BLOCK_N, d), lambda i: (i, 0),
                             pipeline_mode=pl.Buffered(buffer_count=8)),
                pl.BlockSpec((d, h_dim), lambda i: (0, 0)),
                pl.BlockSpec((h_dim, 1), lambda i: (0, 0)),
                pl.BlockSpec((h_dim, q), lambda i: (0, 0)),
                pl.BlockSpec((q, 1), lambda i: (0, 0)),
            ],
            out_specs=[pl.BlockSpec((_BLOCK_N, q), lambda i: (i, 0),
                                    pipeline_mode=pl.Buffered(buffer_count=2))],
            core_axis_name="core",
            dimension_semantics=(pltpu.PARALLEL,),
        )(x_hbm, w1_hbm, b1_hbm, w2_hbm, b2_hbm, o_hbm)

    return pl.kernel(
        body,
        out_type=jax.ShapeDtypeStruct((n, q), jnp.float32),
        mesh=mesh,
    )(batch_x, W1, b1.reshape(h_dim, 1), W2, b2.reshape(q, 1))


# speedup vs baseline: 1.0847x; 1.0168x over previous
import functools

import jax
import jax.numpy as jnp
from jax.experimental import pallas as pl
from jax.experimental.pallas import tpu as pltpu

_BLOCK_N = 4096
_K = 4
_NBUF = 3


def _compute(x, w1, b1, w2, b2):
    ht = jax.lax.dot_general(
        w1, x, (((0,), (1,)), ((), ())),
        preferred_element_type=jnp.float32,
    ) + b1
    ht = 0.5 * ht * (1.0 + jax.lax.erf(ht * 0.7071067811865476))
    lt = jax.lax.dot_general(
        w2, ht, (((0,), (0,)), ((), ())),
        preferred_element_type=jnp.float32,
    ) + b2
    m = jnp.max(lt, axis=0, keepdims=True)
    e = jnp.exp(lt - m)
    return e / jnp.sum(e, axis=0, keepdims=True)


def _body(x_hbm, w1_hbm, b1_hbm, w2_hbm, b2_hbm, o_hbm, *, n, d, h_dim, q):
    spc = n // _BLOCK_N // _K

    def run(w1_v, b1_v, w2_v, b2_v, x_bufs, o_bufs, w_sem, in_sems, out_sems):
        for ref_h, ref_v in ((w1_hbm, w1_v), (b1_hbm, b1_v),
                             (w2_hbm, w2_v), (b2_hbm, b2_v)):
            cp = pltpu.make_async_copy(ref_h, ref_v, w_sem)
            cp.start()
            cp.wait()
        w1 = w1_v[...]
        b1 = b1_v[...]
        w2 = w2_v[...]
        b2 = b2_v[...]

        def in_copy(k, j, slot):
            row0 = (k * spc + j) * _BLOCK_N
            return pltpu.make_async_copy(
                x_hbm.at[pl.ds(row0, _BLOCK_N), :],
                x_bufs.at[k, slot],
                in_sems.at[k, slot],
            )

        def out_copy(k, j, slot):
            row0 = (k * spc + j) * _BLOCK_N
            return pltpu.make_async_copy(
                o_bufs.at[k, slot],
                o_hbm.at[pl.ds(row0, _BLOCK_N), :],
                out_sems.at[k, slot],
            )

        for j in range(_NBUF - 1):
            for k in range(_K):
                in_copy(k, j, j).start()

        def step(j, carry):
            del carry
            slot = jax.lax.rem(j, _NBUF)
            oslot = jax.lax.rem(j, 2)
            for k in range(_K):
                in_copy(k, j, slot).wait()

                @pl.when(j >= 2)
                def _():
                    out_copy(k, j - 2, oslot).wait()

                p = _compute(x_bufs[k, slot], w1, b1, w2, b2)
                o_bufs[k, oslot] = p.T
                out_copy(k, j, oslot).start()

                @pl.when(j + _NBUF - 1 < spc)
                def _():
                    nslot = jax.lax.rem(j + _NBUF - 1, _NBUF)
                    in_copy(k, j + _NBUF - 1, nslot).start()
            return 0

        jax.lax.fori_loop(0, spc, step, 0)

        for k in range(_K):
            for j in (spc - 2, spc - 1):
                out_copy(k, j, j % 2).wait()

    pl.run_scoped(
        run,
        pltpu.VMEM((d, h_dim), jnp.float32),
        pltpu.VMEM((h_dim, 1), jnp.float32),
        pltpu.VMEM((h_dim, q), jnp.float32),
        pltpu.VMEM((q, 1), jnp.float32),
        pltpu.VMEM((_K, _NBUF, _BLOCK_N, d), jnp.float32),
        pltpu.VMEM((_K, 2, _BLOCK_N, q), jnp.float32),
        pltpu.SemaphoreType.DMA,
        pltpu.SemaphoreType.DMA((_K, _NBUF)),
        pltpu.SemaphoreType.DMA((_K, 2)),
    )


def kernel(batch_x, W1, b1, W2, b2):
    n, d = batch_x.shape
    h_dim = W1.shape[1]
    q = W2.shape[1]
    mesh = pltpu.create_tensorcore_mesh("core")
    body = functools.partial(_body, n=n, d=d, h_dim=h_dim, q=q)
    return pl.kernel(
        body,
        out_type=jax.ShapeDtypeStruct((n, q), jnp.float32),
        mesh=mesh,
    )(batch_x, W1, b1.reshape(h_dim, 1), W2, b2.reshape(q, 1))


# speedup vs baseline: 1.9029x; 1.7544x over previous
import functools

import jax
import jax.numpy as jnp
from jax.experimental import pallas as pl
from jax.experimental.pallas import tpu as pltpu

_BLOCK_N = 4096
_K = 4
_NBUF = 3


def _compute(x, w1, b1, w2, b2):
    ht = jax.lax.dot_general(
        w1, x, (((0,), (1,)), ((), ())),
        preferred_element_type=jnp.float32,
    ) + b1
    ht = 0.5 * ht * (1.0 + jax.lax.erf(ht * 0.7071067811865476))
    lt = jax.lax.dot_general(
        w2, ht, (((0,), (0,)), ((), ())),
        preferred_element_type=jnp.float32,
    ) + b2
    m = jnp.max(lt, axis=0, keepdims=True)
    e = jnp.exp(lt - m)
    return e / jnp.sum(e, axis=0, keepdims=True)


def _body(x_hbm, w1_hbm, b1_hbm, w2_hbm, b2_hbm, o_hbm, *, n, d, h_dim, q):
    spc = n // _BLOCK_N // _K

    def run(w1_v, b1_v, w2_v, b2_v, x_bufs, o_bufs, w_sem, in_sems, out_sems):
        for ref_h, ref_v in ((w1_hbm, w1_v), (b1_hbm, b1_v),
                             (w2_hbm, w2_v), (b2_hbm, b2_v)):
            cp = pltpu.make_async_copy(ref_h, ref_v, w_sem)
            cp.start()
            cp.wait()
        w1 = w1_v[...]
        b1 = b1_v[...]
        w2 = w2_v[...]
        b2 = b2_v[...]

        def in_copy(k, j, slot):
            row0 = (k * spc + j) * _BLOCK_N
            return pltpu.make_async_copy(
                x_hbm.at[pl.ds(row0, _BLOCK_N), :],
                x_bufs.at[k, slot],
                in_sems.at[k, slot],
            )

        def out_copy(k, j, slot):
            col0 = (k * spc + j) * _BLOCK_N
            return pltpu.make_async_copy(
                o_bufs.at[k, slot],
                o_hbm.at[:, pl.ds(col0, _BLOCK_N)],
                out_sems.at[k, slot],
            )

        for j in range(_NBUF - 1):
            for k in range(_K):
                in_copy(k, j, j).start()

        def step(j, carry):
            del carry
            slot = jax.lax.rem(j, _NBUF)
            oslot = jax.lax.rem(j, 2)
            for k in range(_K):
                in_copy(k, j, slot).wait()

                @pl.when(j >= 2)
                def _():
                    out_copy(k, j - 2, oslot).wait()

                p = _compute(x_bufs[k, slot], w1, b1, w2, b2)
                o_bufs[k, oslot] = p
                out_copy(k, j, oslot).start()

                @pl.when(j + _NBUF - 1 < spc)
                def _():
                    nslot = jax.lax.rem(j + _NBUF - 1, _NBUF)
                    in_copy(k, j + _NBUF - 1, nslot).start()
            return 0

        jax.lax.fori_loop(0, spc, step, 0)

        for k in range(_K):
            for j in (spc - 2, spc - 1):
                out_copy(k, j, j % 2).wait()

    pl.run_scoped(
        run,
        pltpu.VMEM((d, h_dim), jnp.float32),
        pltpu.VMEM((h_dim, 1), jnp.float32),
        pltpu.VMEM((h_dim, q), jnp.float32),
        pltpu.VMEM((q, 1), jnp.float32),
        pltpu.VMEM((_K, _NBUF, _BLOCK_N, d), jnp.float32),
        pltpu.VMEM((_K, 2, q, _BLOCK_N), jnp.float32),
        pltpu.SemaphoreType.DMA,
        pltpu.SemaphoreType.DMA((_K, _NBUF)),
        pltpu.SemaphoreType.DMA((_K, 2)),
    )


def kernel(batch_x, W1, b1, W2, b2):
    n, d = batch_x.shape
    h_dim = W1.shape[1]
    q = W2.shape[1]
    mesh = pltpu.create_tensorcore_mesh("core")
    body = functools.partial(_body, n=n, d=d, h_dim=h_dim, q=q)
    out_t = pl.kernel(
        body,
        out_type=jax.ShapeDtypeStruct((q, n), jnp.float32),
        mesh=mesh,
    )(batch_x, W1, b1.reshape(h_dim, 1), W2, b2.reshape(q, 1))
    return out_t.T
